# baseline (speedup 1.0000x reference)
import functools

import jax
import jax.numpy as jnp
from jax.experimental import pallas as pl

F32 = jnp.float32


def _relu(x):
    return jnp.maximum(x, 0.0)


def _k0_body(pts_ref, w0p, b0, w1, b1, wmh, wmrp, bm,
             out_ref, xyz_ref, x_ref, y_ref, z_ref, *, rows):
    C = pts_ref.shape[-1]
    pts = pts_ref[...].reshape(rows, C)
    pts_bf = pts.astype(jnp.bfloat16)
    h = _relu(jnp.dot(pts_bf, w0p[...], preferred_element_type=F32) + b0[...])
    h = _relu(jnp.dot(h.astype(jnp.bfloat16), w1[...],
                      preferred_element_type=F32) + b1[...])
    f = _relu(
        jnp.dot(h.astype(jnp.bfloat16), wmh[...], preferred_element_type=F32)
        + jnp.dot(pts_bf, wmrp[...], preferred_element_type=F32)
        + bm[...]
    )
    out_ref[...] = f.reshape(out_ref.shape)
    xyz_ref[...] = pts[:, 0:3]
    x_ref[...] = pts[:, 0:1]
    y_ref[...] = pts[:, 1:2]
    z_ref[...] = pts[:, 2:3]


def _fps_block(xa, ya, za, npoint):
    R, M = xa.shape
    lane = jax.lax.broadcasted_iota(jnp.int32, (R, M), 1)
    col = jax.lax.broadcasted_iota(jnp.int32, (R, npoint), 1)

    px = xa[:, 0:1]
    py = ya[:, 0:1]
    pz = za[:, 0:1]
    cx = jnp.where(col == 0, px, 0.0)
    cy = jnp.where(col == 0, py, 0.0)
    cz = jnp.where(col == 0, pz, 0.0)
    dists = jnp.full((R, M), 1e10, dtype=F32)

    def body(i, st):
        dists, px, py, pz, cx, cy, cz = st
        dx = xa - px
        dy = ya - py
        dz = za - pz
        d = (dx * dx + dy * dy) + dz * dz
        dists = jnp.minimum(dists, d)
        m = jnp.max(dists, axis=1, keepdims=True)
        idx = jnp.min(jnp.where(dists == m, lane, M), axis=1, keepdims=True)
        oh = lane == idx
        nx = jnp.sum(jnp.where(oh, xa, 0.0), axis=1, keepdims=True)
        ny = jnp.sum(jnp.where(oh, ya, 0.0), axis=1, keepdims=True)
        nz = jnp.sum(jnp.where(oh, za, 0.0), axis=1, keepdims=True)
        sel = col == i
        cx = jnp.where(sel, nx, cx)
        cy = jnp.where(sel, ny, cy)
        cz = jnp.where(sel, nz, cz)
        return (dists, nx, ny, nz, cx, cy, cz)

    st = jax.lax.fori_loop(1, npoint, body, (dists, px, py, pz, cx, cy, cz))
    return st[4], st[5], st[6]


def _k1_body(x_ref, y_ref, z_ref,
             c1x_ref, c1y_ref, c1z_ref, c2x_ref, c2y_ref, c2z_ref):
    c1x, c1y, c1z = _fps_block(x_ref[...], y_ref[...], z_ref[...], 128)
    c1x_ref[...] = c1x
    c1y_ref[...] = c1y
    c1z_ref[...] = c1z
    c2x, c2y, c2z = _fps_block(c1x, c1y, c1z, 32)
    c2x_ref[...] = c2x
    c2y_ref[...] = c2y
    c2z_ref[...] = c2z


def _group_stage(cxr, cyr, czr, A, cmat, r2, S,
                 wx, b1, w2, b2, w3, b3):
    P = cmat.shape[0]
    M = cxr.shape[1]
    cxc = cmat[:, 0:1]
    cyc = cmat[:, 1:2]
    czc = cmat[:, 2:3]
    dx = cxc - cxr
    dy = cyc - cyr
    dz = czc - czr
    d2 = (dx * dx + dy * dy) + dz * dz
    maskf = jnp.where(d2 < r2, 1.0, 0.0)

    io_r = jax.lax.broadcasted_iota(jnp.int32, (M, M), 0)
    io_c = jax.lax.broadcasted_iota(jnp.int32, (M, M), 1)
    bf = jnp.bfloat16
    tri = jnp.where(io_r <= io_c, 1.0, 0.0).astype(bf)
    cnt = jnp.dot(maskf.astype(bf), tri, preferred_element_type=F32)
    total = cnt[:, M - 1:M]
    empty = jnp.where(total > 0.0, 0.0, 1.0)
    e0row = jnp.where(
        jax.lax.broadcasted_iota(jnp.int32, (1, M), 1) == 0, 1.0, 0.0)
    maskf = maskf + empty * e0row
    cnt = cnt + empty
    total = total + empty

    mci = (maskf * cnt).astype(jnp.int32)
    s_io1 = jax.lax.broadcasted_iota(jnp.int32, (1, S, 1), 1) + 1
    tgt = jnp.minimum(s_io1, total.astype(jnp.int32)[:, :, None])
    ohf = jnp.where(mci[:, None, :] == tgt, 1.0, 0.0).astype(bf)
    G = ohf.reshape(P * S, M)

    cb = cxc * wx[0:1, :] + cyc * wx[1:2, :] + czc * wx[2:3, :]
    g = jnp.dot(G, A.astype(bf), preferred_element_type=F32).reshape(P, S, -1)
    g = g + (b1 - cb)[:, None, :]
    h = _relu(g).reshape(P * S, -1)
    h = _relu(jnp.dot(h.astype(bf), w2, preferred_element_type=F32) + b2)
    h = _relu(jnp.dot(h.astype(bf), w3, preferred_element_type=F32) + b3)
    C3 = h.shape[-1]
    return jnp.max(h.reshape(P, S, C3), axis=1)


def _k2_body(x_ref, y_ref, z_ref, xyz3_ref, feats_ref,
             c1m_ref, c1x_ref, c1y_ref, c1z_ref, c2m_ref,
             w1x, w1f, b11, w12, b12, w13, b13,
             w2x, w2f, b21, w22, b22, w23, b23,
             out_ref):
    feats = feats_ref[0]
    xyz3 = xyz3_ref[0]
    c1m = c1m_ref[0]
    c2m = c2m_ref[0]
    w2xv = w2x[...]

    A1 = (jnp.dot(xyz3, w1x[...], preferred_element_type=F32)
          + jnp.dot(feats, w1f[...], preferred_element_type=F32))
    f1 = _group_stage(x_ref[0], y_ref[0], z_ref[0], A1, c1m, 0.2 * 0.2, 64,
                      w1x[...], b11[...], w12[...], b12[...], w13[...], b13[...])

    A2 = (c1m[:, 0:1] * w2xv[0:1, :] + c1m[:, 1:2] * w2xv[1:2, :]
          + c1m[:, 2:3] * w2xv[2:3, :]
          + jnp.dot(f1, w2f[...], preferred_element_type=F32))
    f2 = _group_stage(c1x_ref[0], c1y_ref[0], c1z_ref[0], A2, c2m,
                      0.4 * 0.4, 64,
                      w2xv, b21[...], w22[...], b22[...], w23[...], b23[...])
    out_ref[...] = f2[None, :, :]


def _k3_body(xp_ref, fp_ref,
             w3x, w3f, b31, w32, b32, w33, b33,
             wc1, bc1, wc2, bc2, wc3, bc3,
             wr1, br1, wr2, br2, wr3, br3,
             out_ref, *, rois_per_block, npoint):
    w3xv = w3x[...]
    xp = xp_ref[...]
    h = _relu(xp[:, 0:1] * w3xv[0:1, :] + xp[:, 1:2] * w3xv[1:2, :]
              + xp[:, 2:3] * w3xv[2:3, :]
              + jnp.dot(fp_ref[...], w3f[...], preferred_element_type=F32)
              + b31[...])
    h = _relu(jnp.dot(h, w32[...], preferred_element_type=F32) + b32[...])
    h = _relu(jnp.dot(h, w33[...], preferred_element_type=F32) + b33[...])
    f3 = jnp.max(h.reshape(rois_per_block, npoint, h.shape[-1]), axis=1)

    hc = _relu(jnp.dot(f3, wc1[...], preferred_element_type=F32) + bc1[...])
    hc = _relu(jnp.dot(hc, wc2[...], preferred_element_type=F32) + bc2[...])
    c = jnp.dot(hc, wc3[...], preferred_element_type=F32) + bc3[...]

    hr = _relu(jnp.dot(f3, wr1[...], preferred_element_type=F32) + br1[...])
    hr = _relu(jnp.dot(hr, wr2[...], preferred_element_type=F32) + br2[...])
    r = jnp.dot(hr, wr3[...], preferred_element_type=F32) + br3[...]

    out_ref[:, 0:1] = c
    out_ref[:, 1:] = r


def _full_spec(shape):
    n = len(shape)
    return pl.BlockSpec(shape, lambda i, _n=n: (0,) * _n)


def _wt(wb):
    W, b = wb
    return W.T.astype(F32), b.reshape(1, -1).astype(F32)


def kernel(pts_input, roi_boxes3d, params):
    B, N, C = pts_input.shape

    bf = jnp.bfloat16
    w0, b0 = _wt(params['xyz_up'][0])
    w1, b1 = _wt(params['xyz_up'][1])
    Wm, bm_ = params['merge']
    wmh = Wm[:, :128].T
    wmr = Wm[:, 128:].T
    bm = bm_.reshape(1, -1)
    w0p = jnp.zeros((C, 128), F32).at[:5].set(w0).astype(bf)
    wmrp = jnp.zeros((C, 128), F32).at[5:].set(wmr).astype(bf)

    s1 = params['sa1']
    w1x = s1[0][0][:, :3].T
    w1f = s1[0][0][:, 3:].T
    b11 = s1[0][1].reshape(1, -1)
    w12, b12 = _wt(s1[1])
    w13, b13 = _wt(s1[2])

    s2 = params['sa2']
    w2x = s2[0][0][:, :3].T
    w2f = s2[0][0][:, 3:].T
    b21 = s2[0][1].reshape(1, -1)
    w22, b22 = _wt(s2[1])
    w23, b23 = _wt(s2[2])

    s3 = params['sa3']
    w3x = s3[0][0][:, :3].T
    w3f = s3[0][0][:, 3:].T
    b31 = s3[0][1].reshape(1, -1)
    w32, b32 = _wt(s3[1])
    w33, b33 = _wt(s3[2])

    wc1, bc1 = _wt(params['cls'][0])
    wc2, bc2 = _wt(params['cls'][1])
    wc3, bc3 = _wt(params['cls'][2])
    wr1, br1 = _wt(params['reg'][0])
    wr2, br2 = _wt(params['reg'][1])
    wr3, br3 = _wt(params['reg'][2])

    rows = B * N
    rpb0 = 4 if B % 4 == 0 else 1
    blk0 = rpb0 * N
    g0 = B // rpb0
    w1b, wmhb = w1.astype(bf), wmh.astype(bf)
    feats, xyzc, xc, yc, zc = pl.pallas_call(
        functools.partial(_k0_body, rows=blk0),
        grid=(g0,),
        in_specs=[
            pl.BlockSpec((rpb0, N, C), lambda i: (i, 0, 0)),
            _full_spec(w0p.shape), _full_spec(b0.shape),
            _full_spec(w1b.shape), _full_spec(b1.shape),
            _full_spec(wmhb.shape), _full_spec(wmrp.shape),
            _full_spec(bm.shape),
        ],
        out_specs=[
            pl.BlockSpec((rpb0, N, 128), lambda i: (i, 0, 0)),
            pl.BlockSpec((blk0, 3), lambda i: (i, 0)),
            pl.BlockSpec((blk0, 1), lambda i: (i, 0)),
            pl.BlockSpec((blk0, 1), lambda i: (i, 0)),
            pl.BlockSpec((blk0, 1), lambda i: (i, 0)),
        ],
        out_shape=[
            jax.ShapeDtypeStruct((B, N, 128), F32),
            jax.ShapeDtypeStruct((rows, 3), F32),
            jax.ShapeDtypeStruct((rows, 1), F32),
            jax.ShapeDtypeStruct((rows, 1), F32),
            jax.ShapeDtypeStruct((rows, 1), F32),
        ],
    )(pts_input, w0p, b0, w1b, b1, wmhb, wmrp, bm)

    x = xc.reshape(B, N)
    y = yc.reshape(B, N)
    z = zc.reshape(B, N)
    R = B
    g1 = B // R
    fps_out = pl.pallas_call(
        _k1_body,
        grid=(g1,),
        in_specs=[pl.BlockSpec((R, N), lambda i: (i, 0))] * 3,
        out_specs=[pl.BlockSpec((R, 128), lambda i: (i, 0))] * 3
        + [pl.BlockSpec((R, 32), lambda i: (i, 0))] * 3,
        out_shape=[jax.ShapeDtypeStruct((B, 128), F32)] * 3
        + [jax.ShapeDtypeStruct((B, 32), F32)] * 3,
    )(x, y, z)
    c1x, c1y, c1z, c2x, c2y, c2z = fps_out
    c1mat = jnp.stack([c1x, c1y, c1z], axis=-1)
    c2mat = jnp.stack([c2x, c2y, c2z], axis=-1)

    x3 = x.reshape(B, 1, N)
    y3 = y.reshape(B, 1, N)
    z3 = z.reshape(B, 1, N)
    c1x3 = c1x.reshape(B, 1, 128)
    c1y3 = c1y.reshape(B, 1, 128)
    c1z3 = c1z.reshape(B, 1, 128)
    xyz3 = xyzc.reshape(B, N, 3)

    w_in = [w1x, w1f, b11, w12.astype(bf), b12, w13.astype(bf), b13,
            w2x, w2f, b21, w22.astype(bf), b22, w23.astype(bf), b23]
    f2 = pl.pallas_call(
        _k2_body,
        grid=(B,),
        in_specs=[
            pl.BlockSpec((1, 1, N), lambda i: (i, 0, 0)),
            pl.BlockSpec((1, 1, N), lambda i: (i, 0, 0)),
            pl.BlockSpec((1, 1, N), lambda i: (i, 0, 0)),
            pl.BlockSpec((1, N, 3), lambda i: (i, 0, 0)),
            pl.BlockSpec((1, N, 128), lambda i: (i, 0, 0)),
            pl.BlockSpec((1, 128, 3), lambda i: (i, 0, 0)),
            pl.BlockSpec((1, 1, 128), lambda i: (i, 0, 0)),
            pl.BlockSpec((1, 1, 128), lambda i: (i, 0, 0)),
            pl.BlockSpec((1, 1, 128), lambda i: (i, 0, 0)),
            pl.BlockSpec((1, 32, 3), lambda i: (i, 0, 0)),
        ] + [_full_spec(w.shape) for w in w_in],
        out_specs=pl.BlockSpec((1, 32, 256), lambda i: (i, 0, 0)),
        out_shape=jax.ShapeDtypeStruct((B, 32, 256), F32),
    )(x3, y3, z3, xyz3, feats, c1mat, c1x3, c1y3, c1z3, c2mat, *w_in)

    xp = c2mat.reshape(B * 32, 3)
    fp = f2.reshape(B * 32, 256)
    rpb = 64 if B % 64 == 0 else B
    g3 = B // rpb
    w3_in = [w3x, w3f, b31, w32, b32, w33, b33,
             wc1, bc1, wc2, bc2, wc3, bc3,
             wr1, br1, wr2, br2, wr3, br3]
    out = pl.pallas_call(
        functools.partial(_k3_body, rois_per_block=rpb, npoint=32),
        grid=(g3,),
        in_specs=[
            pl.BlockSpec((rpb * 32, 3), lambda i: (i, 0)),
            pl.BlockSpec((rpb * 32, 256), lambda i: (i, 0)),
        ] + [_full_spec(w.shape) for w in w3_in],
        out_specs=pl.BlockSpec((rpb, 47), lambda i: (i, 0)),
        out_shape=jax.ShapeDtypeStruct((B, 47), F32),
    )(xp, fp, *w3_in)
    return out


# speedup vs baseline: 1.0067x; 1.0067x over previous
import functools

import jax
import jax.numpy as jnp
from jax.experimental import pallas as pl

F32 = jnp.float32


def _relu(x):
    return jnp.maximum(x, 0.0)


def _k0_body(pts_ref, w0p, b0, w1, b1, wmh, wmrp, bm,
             out_ref, x_ref, y_ref, z_ref, *, rows):
    C = pts_ref.shape[-1]
    pts = pts_ref[...].reshape(rows, C)
    pts_bf = pts.astype(jnp.bfloat16)
    h = _relu(jnp.dot(pts_bf, w0p[...], preferred_element_type=F32) + b0[...])
    h = _relu(jnp.dot(h.astype(jnp.bfloat16), w1[...],
                      preferred_element_type=F32) + b1[...])
    f = _relu(
        jnp.dot(h.astype(jnp.bfloat16), wmh[...], preferred_element_type=F32)
        + jnp.dot(pts_bf, wmrp[...], preferred_element_type=F32)
        + bm[...]
    )
    out_ref[...] = f.reshape(out_ref.shape)
    x_ref[...] = pts[:, 0:1]
    y_ref[...] = pts[:, 1:2]
    z_ref[...] = pts[:, 2:3]


def _fps_block(xa, ya, za, npoint):
    R, M = xa.shape
    lane = jax.lax.broadcasted_iota(jnp.int32, (R, M), 1)
    col = jax.lax.broadcasted_iota(jnp.int32, (R, npoint), 1)

    px = xa[:, 0:1]
    py = ya[:, 0:1]
    pz = za[:, 0:1]
    cx = jnp.where(col == 0, px, 0.0)
    cy = jnp.where(col == 0, py, 0.0)
    cz = jnp.where(col == 0, pz, 0.0)
    dists = jnp.full((R, M), 1e10, dtype=F32)

    def body(i, st):
        dists, px, py, pz, cx, cy, cz = st
        dx = xa - px
        dy = ya - py
        dz = za - pz
        d = (dx * dx + dy * dy) + dz * dz
        dists = jnp.minimum(dists, d)
        m = jnp.max(dists, axis=1, keepdims=True)
        idx = jnp.min(jnp.where(dists == m, lane, M), axis=1, keepdims=True)
        oh = lane == idx
        nx = jnp.sum(jnp.where(oh, xa, 0.0), axis=1, keepdims=True)
        ny = jnp.sum(jnp.where(oh, ya, 0.0), axis=1, keepdims=True)
        nz = jnp.sum(jnp.where(oh, za, 0.0), axis=1, keepdims=True)
        sel = col == i
        cx = jnp.where(sel, nx, cx)
        cy = jnp.where(sel, ny, cy)
        cz = jnp.where(sel, nz, cz)
        return (dists, nx, ny, nz, cx, cy, cz)

    st = jax.lax.fori_loop(1, npoint, body, (dists, px, py, pz, cx, cy, cz))
    return st[4], st[5], st[6]


def _k1_body(x_ref, y_ref, z_ref,
             c1x_ref, c1y_ref, c1z_ref, c2x_ref, c2y_ref, c2z_ref):
    c1x, c1y, c1z = _fps_block(x_ref[...], y_ref[...], z_ref[...], 128)
    c1x_ref[...] = c1x
    c1y_ref[...] = c1y
    c1z_ref[...] = c1z
    c2x, c2y, c2z = _fps_block(c1x, c1y, c1z, 32)
    c2x_ref[...] = c2x
    c2y_ref[...] = c2y
    c2z_ref[...] = c2z


def _group_stage(cxr, cyr, czr, A, cmat, r2, S,
                 wx, b1, w2, b2, w3, b3):
    P = cmat.shape[0]
    M = cxr.shape[1]
    cxc = cmat[:, 0:1]
    cyc = cmat[:, 1:2]
    czc = cmat[:, 2:3]
    dx = cxc - cxr
    dy = cyc - cyr
    dz = czc - czr
    d2 = (dx * dx + dy * dy) + dz * dz
    maskf = jnp.where(d2 < r2, 1.0, 0.0)

    io_r = jax.lax.broadcasted_iota(jnp.int32, (M, M), 0)
    io_c = jax.lax.broadcasted_iota(jnp.int32, (M, M), 1)
    bf = jnp.bfloat16
    tri = jnp.where(io_r <= io_c, 1.0, 0.0).astype(bf)
    cnt = jnp.dot(maskf.astype(bf), tri, preferred_element_type=F32)
    total = cnt[:, M - 1:M]
    empty = jnp.where(total > 0.0, 0.0, 1.0)
    e0row = jnp.where(
        jax.lax.broadcasted_iota(jnp.int32, (1, M), 1) == 0, 1.0, 0.0)
    maskf = maskf + empty * e0row
    cnt = cnt + empty
    total = total + empty

    mci = (maskf * cnt).astype(jnp.int32)
    s_io1 = jax.lax.broadcasted_iota(jnp.int32, (1, S, 1), 1) + 1
    ohf = jnp.where(mci[:, None, :] == s_io1, 1.0, 0.0).astype(bf)
    G = ohf.reshape(P * S, M)

    cb = cxc * wx[0:1, :] + cyc * wx[1:2, :] + czc * wx[2:3, :]
    g = jnp.dot(G, A.astype(bf), preferred_element_type=F32).reshape(P, S, -1)
    g = g + (b1 - cb)[:, None, :]
    h = _relu(g).reshape(P * S, -1)
    h = _relu(jnp.dot(h.astype(bf), w2, preferred_element_type=F32) + b2)
    h = _relu(jnp.dot(h.astype(bf), w3, preferred_element_type=F32) + b3)
    C3 = h.shape[-1]
    pen = jnp.where(s_io1 <= total[:, :, None].astype(jnp.int32), 0.0, -1e30)
    return jnp.max(h.reshape(P, S, C3) + pen, axis=1)


def _k2_body(x_ref, y_ref, z_ref, feats_ref,
             c1m_ref, c1x_ref, c1y_ref, c1z_ref, c2m_ref,
             w1x, w1f, b11, w12, b12, w13, b13,
             w2x, w2f, b21, w22, b22, w23, b23,
             out_ref):
    feats = feats_ref[0]
    c1m = c1m_ref[0]
    c2m = c2m_ref[0]
    w2xv = w2x[...]

    xyzT = jnp.concatenate([x_ref[0], y_ref[0], z_ref[0]], axis=0)
    A1 = (jax.lax.dot_general(xyzT, w1x[...], (((0,), (0,)), ((), ())),
                              preferred_element_type=F32)
          + jnp.dot(feats, w1f[...], preferred_element_type=F32))
    f1 = _group_stage(x_ref[0], y_ref[0], z_ref[0], A1, c1m, 0.2 * 0.2, 64,
                      w1x[...], b11[...], w12[...], b12[...], w13[...], b13[...])

    A2 = (c1m[:, 0:1] * w2xv[0:1, :] + c1m[:, 1:2] * w2xv[1:2, :]
          + c1m[:, 2:3] * w2xv[2:3, :]
          + jnp.dot(f1, w2f[...], preferred_element_type=F32))
    f2 = _group_stage(c1x_ref[0], c1y_ref[0], c1z_ref[0], A2, c2m,
                      0.4 * 0.4, 64,
                      w2xv, b21[...], w22[...], b22[...], w23[...], b23[...])
    out_ref[...] = f2[None, :, :]


def _k3_body(xp_ref, fp_ref,
             w3x, w3f, b31, w32, b32, w33, b33,
             wc1, bc1, wc2, bc2, wc3, bc3,
             wr1, br1, wr2, br2, wr3, br3,
             out_ref, *, rois_per_block, npoint):
    w3xv = w3x[...]
    xp = xp_ref[...]
    h = _relu(xp[:, 0:1] * w3xv[0:1, :] + xp[:, 1:2] * w3xv[1:2, :]
              + xp[:, 2:3] * w3xv[2:3, :]
              + jnp.dot(fp_ref[...], w3f[...], preferred_element_type=F32)
              + b31[...])
    h = _relu(jnp.dot(h, w32[...], preferred_element_type=F32) + b32[...])
    h = _relu(jnp.dot(h, w33[...], preferred_element_type=F32) + b33[...])
    f3 = jnp.max(h.reshape(rois_per_block, npoint, h.shape[-1]), axis=1)

    hc = _relu(jnp.dot(f3, wc1[...], preferred_element_type=F32) + bc1[...])
    hc = _relu(jnp.dot(hc, wc2[...], preferred_element_type=F32) + bc2[...])
    c = jnp.dot(hc, wc3[...], preferred_element_type=F32) + bc3[...]

    hr = _relu(jnp.dot(f3, wr1[...], preferred_element_type=F32) + br1[...])
    hr = _relu(jnp.dot(hr, wr2[...], preferred_element_type=F32) + br2[...])
    r = jnp.dot(hr, wr3[...], preferred_element_type=F32) + br3[...]

    out_ref[:, 0:1] = c
    out_ref[:, 1:] = r


def _full_spec(shape):
    n = len(shape)
    return pl.BlockSpec(shape, lambda i, _n=n: (0,) * _n)


def _wt(wb):
    W, b = wb
    return W.T.astype(F32), b.reshape(1, -1).astype(F32)


def kernel(pts_input, roi_boxes3d, params):
    B, N, C = pts_input.shape

    bf = jnp.bfloat16
    w0, b0 = _wt(params['xyz_up'][0])
    w1, b1 = _wt(params['xyz_up'][1])
    Wm, bm_ = params['merge']
    wmh = Wm[:, :128].T
    wmr = Wm[:, 128:].T
    bm = bm_.reshape(1, -1)
    w0p = jnp.zeros((C, 128), F32).at[:5].set(w0).astype(bf)
    wmrp = jnp.zeros((C, 128), F32).at[5:].set(wmr).astype(bf)

    s1 = params['sa1']
    w1x = s1[0][0][:, :3].T
    w1f = s1[0][0][:, 3:].T
    b11 = s1[0][1].reshape(1, -1)
    w12, b12 = _wt(s1[1])
    w13, b13 = _wt(s1[2])

    s2 = params['sa2']
    w2x = s2[0][0][:, :3].T
    w2f = s2[0][0][:, 3:].T
    b21 = s2[0][1].reshape(1, -1)
    w22, b22 = _wt(s2[1])
    w23, b23 = _wt(s2[2])

    s3 = params['sa3']
    w3x = s3[0][0][:, :3].T
    w3f = s3[0][0][:, 3:].T
    b31 = s3[0][1].reshape(1, -1)
    w32, b32 = _wt(s3[1])
    w33, b33 = _wt(s3[2])

    wc1, bc1 = _wt(params['cls'][0])
    wc2, bc2 = _wt(params['cls'][1])
    wc3, bc3 = _wt(params['cls'][2])
    wr1, br1 = _wt(params['reg'][0])
    wr2, br2 = _wt(params['reg'][1])
    wr3, br3 = _wt(params['reg'][2])

    rows = B * N
    rpb0 = 4 if B % 4 == 0 else 1
    blk0 = rpb0 * N
    g0 = B // rpb0
    w1b, wmhb = w1.astype(bf), wmh.astype(bf)
    feats, xc, yc, zc = pl.pallas_call(
        functools.partial(_k0_body, rows=blk0),
        grid=(g0,),
        in_specs=[
            pl.BlockSpec((rpb0, N, C), lambda i: (i, 0, 0)),
            _full_spec(w0p.shape), _full_spec(b0.shape),
            _full_spec(w1b.shape), _full_spec(b1.shape),
            _full_spec(wmhb.shape), _full_spec(wmrp.shape),
            _full_spec(bm.shape),
        ],
        out_specs=[
            pl.BlockSpec((rpb0, N, 128), lambda i: (i, 0, 0)),
            pl.BlockSpec((blk0, 1), lambda i: (i, 0)),
            pl.BlockSpec((blk0, 1), lambda i: (i, 0)),
            pl.BlockSpec((blk0, 1), lambda i: (i, 0)),
        ],
        out_shape=[
            jax.ShapeDtypeStruct((B, N, 128), F32),
            jax.ShapeDtypeStruct((rows, 1), F32),
            jax.ShapeDtypeStruct((rows, 1), F32),
            jax.ShapeDtypeStruct((rows, 1), F32),
        ],
    )(pts_input, w0p, b0, w1b, b1, wmhb, wmrp, bm)

    x = xc.reshape(B, N)
    y = yc.reshape(B, N)
    z = zc.reshape(B, N)
    R = B
    g1 = B // R
    fps_out = pl.pallas_call(
        _k1_body,
        grid=(g1,),
        in_specs=[pl.BlockSpec((R, N), lambda i: (i, 0))] * 3,
        out_specs=[pl.BlockSpec((R, 128), lambda i: (i, 0))] * 3
        + [pl.BlockSpec((R, 32), lambda i: (i, 0))] * 3,
        out_shape=[jax.ShapeDtypeStruct((B, 128), F32)] * 3
        + [jax.ShapeDtypeStruct((B, 32), F32)] * 3,
    )(x, y, z)
    c1x, c1y, c1z, c2x, c2y, c2z = fps_out
    c1mat = jnp.stack([c1x, c1y, c1z], axis=-1)
    c2mat = jnp.stack([c2x, c2y, c2z], axis=-1)

    x3 = x.reshape(B, 1, N)
    y3 = y.reshape(B, 1, N)
    z3 = z.reshape(B, 1, N)
    c1x3 = c1x.reshape(B, 1, 128)
    c1y3 = c1y.reshape(B, 1, 128)
    c1z3 = c1z.reshape(B, 1, 128)

    w_in = [w1x, w1f, b11, w12.astype(bf), b12, w13.astype(bf), b13,
            w2x, w2f, b21, w22.astype(bf), b22, w23.astype(bf), b23]
    f2 = pl.pallas_call(
        _k2_body,
        grid=(B,),
        in_specs=[
            pl.BlockSpec((1, 1, N), lambda i: (i, 0, 0)),
            pl.BlockSpec((1, 1, N), lambda i: (i, 0, 0)),
            pl.BlockSpec((1, 1, N), lambda i: (i, 0, 0)),
            pl.BlockSpec((1, N, 128), lambda i: (i, 0, 0)),
            pl.BlockSpec((1, 128, 3), lambda i: (i, 0, 0)),
            pl.BlockSpec((1, 1, 128), lambda i: (i, 0, 0)),
            pl.BlockSpec((1, 1, 128), lambda i: (i, 0, 0)),
            pl.BlockSpec((1, 1, 128), lambda i: (i, 0, 0)),
            pl.BlockSpec((1, 32, 3), lambda i: (i, 0, 0)),
        ] + [_full_spec(w.shape) for w in w_in],
        out_specs=pl.BlockSpec((1, 32, 256), lambda i: (i, 0, 0)),
        out_shape=jax.ShapeDtypeStruct((B, 32, 256), F32),
    )(x3, y3, z3, feats, c1mat, c1x3, c1y3, c1z3, c2mat, *w_in)

    xp = c2mat.reshape(B * 32, 3)
    fp = f2.reshape(B * 32, 256)
    rpb = 64 if B % 64 == 0 else B
    g3 = B // rpb
    w3_in = [w3x, w3f, b31, w32, b32, w33, b33,
             wc1, bc1, wc2, bc2, wc3, bc3,
             wr1, br1, wr2, br2, wr3, br3]
    out = pl.pallas_call(
        functools.partial(_k3_body, rois_per_block=rpb, npoint=32),
        grid=(g3,),
        in_specs=[
            pl.BlockSpec((rpb * 32, 3), lambda i: (i, 0)),
            pl.BlockSpec((rpb * 32, 256), lambda i: (i, 0)),
        ] + [_full_spec(w.shape) for w in w3_in],
        out_specs=pl.BlockSpec((rpb, 47), lambda i: (i, 0)),
        out_shape=jax.ShapeDtypeStruct((B, 47), F32),
    )(xp, fp, *w3_in)
    return out


# speedup vs baseline: 1.0078x; 1.0011x over previous
import functools

import jax
import jax.numpy as jnp
from jax.experimental import pallas as pl

F32 = jnp.float32


def _relu(x):
    return jnp.maximum(x, 0.0)


def _k0_body(pts_ref, w0p, b0, w1, b1, wmh, wmrp, bm,
             out_ref, *, rows):
    C = pts_ref.shape[-1]
    pts = pts_ref[...].reshape(rows, C)
    pts_bf = pts.astype(jnp.bfloat16)
    h = _relu(jnp.dot(pts_bf, w0p[...], preferred_element_type=F32) + b0[...])
    h = _relu(jnp.dot(h.astype(jnp.bfloat16), w1[...],
                      preferred_element_type=F32) + b1[...])
    f = _relu(
        jnp.dot(h.astype(jnp.bfloat16), wmh[...], preferred_element_type=F32)
        + jnp.dot(pts_bf, wmrp[...], preferred_element_type=F32)
        + bm[...]
    )
    out_ref[...] = f.reshape(out_ref.shape)


def _fps_block(xa, ya, za, npoint):
    R, M = xa.shape
    lane = jax.lax.broadcasted_iota(jnp.int32, (R, M), 1)
    col = jax.lax.broadcasted_iota(jnp.int32, (R, npoint), 1)

    px = xa[:, 0:1]
    py = ya[:, 0:1]
    pz = za[:, 0:1]
    cx = jnp.where(col == 0, px, 0.0)
    cy = jnp.where(col == 0, py, 0.0)
    cz = jnp.where(col == 0, pz, 0.0)
    dists = jnp.full((R, M), 1e10, dtype=F32)

    def body(i, st):
        dists, px, py, pz, cx, cy, cz = st
        dx = xa - px
        dy = ya - py
        dz = za - pz
        d = (dx * dx + dy * dy) + dz * dz
        dists = jnp.minimum(dists, d)
        m = jnp.max(dists, axis=1, keepdims=True)
        idx = jnp.min(jnp.where(dists == m, lane, M), axis=1, keepdims=True)
        oh = lane == idx
        nx = jnp.sum(jnp.where(oh, xa, 0.0), axis=1, keepdims=True)
        ny = jnp.sum(jnp.where(oh, ya, 0.0), axis=1, keepdims=True)
        nz = jnp.sum(jnp.where(oh, za, 0.0), axis=1, keepdims=True)
        sel = col == i
        cx = jnp.where(sel, nx, cx)
        cy = jnp.where(sel, ny, cy)
        cz = jnp.where(sel, nz, cz)
        return (dists, nx, ny, nz, cx, cy, cz)

    st = jax.lax.fori_loop(1, npoint, body, (dists, px, py, pz, cx, cy, cz))
    return st[4], st[5], st[6]


def _k1_body(x_ref, y_ref, z_ref,
             c1x_ref, c1y_ref, c1z_ref, c2x_ref, c2y_ref, c2z_ref):
    c1x, c1y, c1z = _fps_block(x_ref[...], y_ref[...], z_ref[...], 128)
    c1x_ref[...] = c1x
    c1y_ref[...] = c1y
    c1z_ref[...] = c1z
    c2x, c2y, c2z = _fps_block(c1x, c1y, c1z, 32)
    c2x_ref[...] = c2x
    c2y_ref[...] = c2y
    c2z_ref[...] = c2z


def _group_stage(cxr, cyr, czr, A, cmat, r2, S,
                 wx, b1, w2, b2, w3, b3):
    P = cmat.shape[0]
    M = cxr.shape[1]
    cxc = cmat[:, 0:1]
    cyc = cmat[:, 1:2]
    czc = cmat[:, 2:3]
    dx = cxc - cxr
    dy = cyc - cyr
    dz = czc - czr
    d2 = (dx * dx + dy * dy) + dz * dz
    maskf = jnp.where(d2 < r2, 1.0, 0.0)

    io_r = jax.lax.broadcasted_iota(jnp.int32, (M, M), 0)
    io_c = jax.lax.broadcasted_iota(jnp.int32, (M, M), 1)
    bf = jnp.bfloat16
    tri = jnp.where(io_r <= io_c, 1.0, 0.0).astype(bf)
    cnt = jnp.dot(maskf.astype(bf), tri, preferred_element_type=F32)
    total = cnt[:, M - 1:M]
    empty = jnp.where(total > 0.0, 0.0, 1.0)
    e0row = jnp.where(
        jax.lax.broadcasted_iota(jnp.int32, (1, M), 1) == 0, 1.0, 0.0)
    maskf = maskf + empty * e0row
    cnt = cnt + empty
    total = total + empty

    mci = (maskf * cnt).astype(jnp.int32)
    s_io1 = jax.lax.broadcasted_iota(jnp.int32, (1, S, 1), 1) + 1
    ohf = jnp.where(mci[:, None, :] == s_io1, 1.0, 0.0).astype(bf)
    G = ohf.reshape(P * S, M)

    cb = cxc * wx[0:1, :] + cyc * wx[1:2, :] + czc * wx[2:3, :]
    g = jnp.dot(G, A.astype(bf), preferred_element_type=F32).reshape(P, S, -1)
    g = g + (b1 - cb)[:, None, :]
    h = _relu(g).reshape(P * S, -1)
    h = _relu(jnp.dot(h.astype(bf), w2, preferred_element_type=F32) + b2)
    h = _relu(jnp.dot(h.astype(bf), w3, preferred_element_type=F32) + b3)
    C3 = h.shape[-1]
    pen = jnp.where(s_io1 <= total[:, :, None].astype(jnp.int32), 0.0, -1e30)
    return jnp.max(h.reshape(P, S, C3) + pen, axis=1)


def _k2_body(x_ref, y_ref, z_ref, xyz3_ref, feats_ref,
             c1m_ref, c1x_ref, c1y_ref, c1z_ref, c2m_ref,
             w1x, w1f, b11, w12, b12, w13, b13,
             w2x, w2f, b21, w22, b22, w23, b23,
             out_ref):
    feats = feats_ref[0]
    xyz3 = xyz3_ref[0]
    c1m = c1m_ref[0]
    c2m = c2m_ref[0]
    w2xv = w2x[...]

    A1 = (jnp.dot(xyz3, w1x[...], preferred_element_type=F32)
          + jnp.dot(feats, w1f[...], preferred_element_type=F32))
    f1 = _group_stage(x_ref[0], y_ref[0], z_ref[0], A1, c1m, 0.2 * 0.2, 64,
                      w1x[...], b11[...], w12[...], b12[...], w13[...], b13[...])

    A2 = (c1m[:, 0:1] * w2xv[0:1, :] + c1m[:, 1:2] * w2xv[1:2, :]
          + c1m[:, 2:3] * w2xv[2:3, :]
          + jnp.dot(f1, w2f[...], preferred_element_type=F32))
    f2 = _group_stage(c1x_ref[0], c1y_ref[0], c1z_ref[0], A2, c2m,
                      0.4 * 0.4, 64,
                      w2xv, b21[...], w22[...], b22[...], w23[...], b23[...])
    out_ref[...] = f2[None, :, :]


def _k3_body(xp_ref, fp_ref,
             w3x, w3f, b31, w32, b32, w33, b33,
             wc1, bc1, wc2, bc2, wc3, bc3,
             wr1, br1, wr2, br2, wr3, br3,
             out_ref, *, rois_per_block, npoint):
    w3xv = w3x[...]
    xp = xp_ref[...]
    h = _relu(xp[:, 0:1] * w3xv[0:1, :] + xp[:, 1:2] * w3xv[1:2, :]
              + xp[:, 2:3] * w3xv[2:3, :]
              + jnp.dot(fp_ref[...], w3f[...], preferred_element_type=F32)
              + b31[...])
    h = _relu(jnp.dot(h, w32[...], preferred_element_type=F32) + b32[...])
    h = _relu(jnp.dot(h, w33[...], preferred_element_type=F32) + b33[...])
    f3 = jnp.max(h.reshape(rois_per_block, npoint, h.shape[-1]), axis=1)

    hc = _relu(jnp.dot(f3, wc1[...], preferred_element_type=F32) + bc1[...])
    hc = _relu(jnp.dot(hc, wc2[...], preferred_element_type=F32) + bc2[...])
    c = jnp.dot(hc, wc3[...], preferred_element_type=F32) + bc3[...]

    hr = _relu(jnp.dot(f3, wr1[...], preferred_element_type=F32) + br1[...])
    hr = _relu(jnp.dot(hr, wr2[...], preferred_element_type=F32) + br2[...])
    r = jnp.dot(hr, wr3[...], preferred_element_type=F32) + br3[...]

    out_ref[:, 0:1] = c
    out_ref[:, 1:] = r


def _full_spec(shape):
    n = len(shape)
    return pl.BlockSpec(shape, lambda i, _n=n: (0,) * _n)


def _wt(wb):
    W, b = wb
    return W.T.astype(F32), b.reshape(1, -1).astype(F32)


def kernel(pts_input, roi_boxes3d, params):
    B, N, C = pts_input.shape

    bf = jnp.bfloat16
    w0, b0 = _wt(params['xyz_up'][0])
    w1, b1 = _wt(params['xyz_up'][1])
    Wm, bm_ = params['merge']
    wmh = Wm[:, :128].T
    wmr = Wm[:, 128:].T
    bm = bm_.reshape(1, -1)
    w0p = jnp.zeros((C, 128), F32).at[:5].set(w0).astype(bf)
    wmrp = jnp.zeros((C, 128), F32).at[5:].set(wmr).astype(bf)

    s1 = params['sa1']
    w1x = s1[0][0][:, :3].T
    w1f = s1[0][0][:, 3:].T
    b11 = s1[0][1].reshape(1, -1)
    w12, b12 = _wt(s1[1])
    w13, b13 = _wt(s1[2])

    s2 = params['sa2']
    w2x = s2[0][0][:, :3].T
    w2f = s2[0][0][:, 3:].T
    b21 = s2[0][1].reshape(1, -1)
    w22, b22 = _wt(s2[1])
    w23, b23 = _wt(s2[2])

    s3 = params['sa3']
    w3x = s3[0][0][:, :3].T
    w3f = s3[0][0][:, 3:].T
    b31 = s3[0][1].reshape(1, -1)
    w32, b32 = _wt(s3[1])
    w33, b33 = _wt(s3[2])

    wc1, bc1 = _wt(params['cls'][0])
    wc2, bc2 = _wt(params['cls'][1])
    wc3, bc3 = _wt(params['cls'][2])
    wr1, br1 = _wt(params['reg'][0])
    wr2, br2 = _wt(params['reg'][1])
    wr3, br3 = _wt(params['reg'][2])

    rows = B * N
    rpb0 = 4 if B % 4 == 0 else 1
    blk0 = rpb0 * N
    g0 = B // rpb0
    w1b, wmhb = w1.astype(bf), wmh.astype(bf)
    feats = pl.pallas_call(
        functools.partial(_k0_body, rows=blk0),
        grid=(g0,),
        in_specs=[
            pl.BlockSpec((rpb0, N, C), lambda i: (i, 0, 0)),
            _full_spec(w0p.shape), _full_spec(b0.shape),
            _full_spec(w1b.shape), _full_spec(b1.shape),
            _full_spec(wmhb.shape), _full_spec(wmrp.shape),
            _full_spec(bm.shape),
        ],
        out_specs=pl.BlockSpec((rpb0, N, 128), lambda i: (i, 0, 0)),
        out_shape=jax.ShapeDtypeStruct((B, N, 128), F32),
    )(pts_input, w0p, b0, w1b, b1, wmhb, wmrp, bm)

    x = pts_input[:, :, 0]
    y = pts_input[:, :, 1]
    z = pts_input[:, :, 2]
    xyz3 = pts_input[:, :, 0:3]
    R = B
    g1 = B // R
    fps_out = pl.pallas_call(
        _k1_body,
        grid=(g1,),
        in_specs=[pl.BlockSpec((R, N), lambda i: (i, 0))] * 3,
        out_specs=[pl.BlockSpec((R, 128), lambda i: (i, 0))] * 3
        + [pl.BlockSpec((R, 32), lambda i: (i, 0))] * 3,
        out_shape=[jax.ShapeDtypeStruct((B, 128), F32)] * 3
        + [jax.ShapeDtypeStruct((B, 32), F32)] * 3,
    )(x, y, z)
    c1x, c1y, c1z, c2x, c2y, c2z = fps_out
    c1mat = jnp.stack([c1x, c1y, c1z], axis=-1)
    c2mat = jnp.stack([c2x, c2y, c2z], axis=-1)

    x3 = x.reshape(B, 1, N)
    y3 = y.reshape(B, 1, N)
    z3 = z.reshape(B, 1, N)
    c1x3 = c1x.reshape(B, 1, 128)
    c1y3 = c1y.reshape(B, 1, 128)
    c1z3 = c1z.reshape(B, 1, 128)

    w_in = [w1x, w1f, b11, w12.astype(bf), b12, w13.astype(bf), b13,
            w2x, w2f, b21, w22.astype(bf), b22, w23.astype(bf), b23]
    f2 = pl.pallas_call(
        _k2_body,
        grid=(B,),
        in_specs=[
            pl.BlockSpec((1, 1, N), lambda i: (i, 0, 0)),
            pl.BlockSpec((1, 1, N), lambda i: (i, 0, 0)),
            pl.BlockSpec((1, 1, N), lambda i: (i, 0, 0)),
            pl.BlockSpec((1, N, 3), lambda i: (i, 0, 0)),
            pl.BlockSpec((1, N, 128), lambda i: (i, 0, 0)),
            pl.BlockSpec((1, 128, 3), lambda i: (i, 0, 0)),
            pl.BlockSpec((1, 1, 128), lambda i: (i, 0, 0)),
            pl.BlockSpec((1, 1, 128), lambda i: (i, 0, 0)),
            pl.BlockSpec((1, 1, 128), lambda i: (i, 0, 0)),
            pl.BlockSpec((1, 32, 3), lambda i: (i, 0, 0)),
        ] + [_full_spec(w.shape) for w in w_in],
        out_specs=pl.BlockSpec((1, 32, 256), lambda i: (i, 0, 0)),
        out_shape=jax.ShapeDtypeStruct((B, 32, 256), F32),
    )(x3, y3, z3, xyz3, feats, c1mat, c1x3, c1y3, c1z3, c2mat, *w_in)

    xp = c2mat.reshape(B * 32, 3)
    fp = f2.reshape(B * 32, 256)
    rpb = 64 if B % 64 == 0 else B
    g3 = B // rpb
    w3_in = [w3x, w3f, b31, w32, b32, w33, b33,
             wc1, bc1, wc2, bc2, wc3, bc3,
             wr1, br1, wr2, br2, wr3, br3]
    out = pl.pallas_call(
        functools.partial(_k3_body, rois_per_block=rpb, npoint=32),
        grid=(g3,),
        in_specs=[
            pl.BlockSpec((rpb * 32, 3), lambda i: (i, 0)),
            pl.BlockSpec((rpb * 32, 256), lambda i: (i, 0)),
        ] + [_full_spec(w.shape) for w in w3_in],
        out_specs=pl.BlockSpec((rpb, 47), lambda i: (i, 0)),
        out_shape=jax.ShapeDtypeStruct((B, 47), F32),
    )(xp, fp, *w3_in)
    return out


# speedup vs baseline: 1.0681x; 1.0598x over previous
import functools

import jax
import jax.numpy as jnp
from jax.experimental import pallas as pl

F32 = jnp.float32


def _relu(x):
    return jnp.maximum(x, 0.0)


def _k0_body(pts_ref, w0p, b0, w1, b1, wmh, wmrp, bm,
             out_ref, *, rows):
    C = pts_ref.shape[-1]
    pts = pts_ref[...].reshape(rows, C)
    pts_bf = pts.astype(jnp.bfloat16)
    h = _relu(jnp.dot(pts_bf, w0p[...], preferred_element_type=F32) + b0[...])
    h = _relu(jnp.dot(h.astype(jnp.bfloat16), w1[...],
                      preferred_element_type=F32) + b1[...])
    f = _relu(
        jnp.dot(h.astype(jnp.bfloat16), wmh[...], preferred_element_type=F32)
        + jnp.dot(pts_bf, wmrp[...], preferred_element_type=F32)
        + bm[...]
    )
    out_ref[...] = f.astype(jnp.bfloat16).reshape(out_ref.shape)


def _fps_block(xa, ya, za, npoint):
    R, M = xa.shape
    lane = jax.lax.broadcasted_iota(jnp.int32, (R, M), 1)
    col = jax.lax.broadcasted_iota(jnp.int32, (R, npoint), 1)

    px = xa[:, 0:1]
    py = ya[:, 0:1]
    pz = za[:, 0:1]
    cx = jnp.where(col == 0, px, 0.0)
    cy = jnp.where(col == 0, py, 0.0)
    cz = jnp.where(col == 0, pz, 0.0)
    dists = jnp.full((R, M), 1e10, dtype=F32)

    def body(i, st):
        dists, px, py, pz, cx, cy, cz = st
        dx = xa - px
        dy = ya - py
        dz = za - pz
        d = (dx * dx + dy * dy) + dz * dz
        dists = jnp.minimum(dists, d)
        m = jnp.max(dists, axis=1, keepdims=True)
        idx = jnp.min(jnp.where(dists == m, lane, M), axis=1, keepdims=True)
        oh = lane == idx
        nx = jnp.sum(jnp.where(oh, xa, 0.0), axis=1, keepdims=True)
        ny = jnp.sum(jnp.where(oh, ya, 0.0), axis=1, keepdims=True)
        nz = jnp.sum(jnp.where(oh, za, 0.0), axis=1, keepdims=True)
        sel = col == i
        cx = jnp.where(sel, nx, cx)
        cy = jnp.where(sel, ny, cy)
        cz = jnp.where(sel, nz, cz)
        return (dists, nx, ny, nz, cx, cy, cz)

    st = jax.lax.fori_loop(1, npoint, body, (dists, px, py, pz, cx, cy, cz))
    return st[4], st[5], st[6]


def _k1_body(x_ref, y_ref, z_ref,
             c1x_ref, c1y_ref, c1z_ref, c2x_ref, c2y_ref, c2z_ref):
    c1x, c1y, c1z = _fps_block(x_ref[...], y_ref[...], z_ref[...], 128)
    c1x_ref[...] = c1x
    c1y_ref[...] = c1y
    c1z_ref[...] = c1z
    c2x, c2y, c2z = _fps_block(c1x, c1y, c1z, 32)
    c2x_ref[...] = c2x
    c2y_ref[...] = c2y
    c2z_ref[...] = c2z


def _group_stage(cxr, cyr, czr, A, cmat, r2, S,
                 wx, b1, w2, b2, w3, b3):
    P = cmat.shape[0]
    M = cxr.shape[1]
    cxc = cmat[:, 0:1]
    cyc = cmat[:, 1:2]
    czc = cmat[:, 2:3]
    dx = cxc - cxr
    dy = cyc - cyr
    dz = czc - czr
    d2 = (dx * dx + dy * dy) + dz * dz
    maskf = jnp.where(d2 < r2, 1.0, 0.0)

    io_r = jax.lax.broadcasted_iota(jnp.int32, (M, M), 0)
    io_c = jax.lax.broadcasted_iota(jnp.int32, (M, M), 1)
    bf = jnp.bfloat16
    tri = jnp.where(io_r <= io_c, 1.0, 0.0).astype(bf)
    cnt = jnp.dot(maskf.astype(bf), tri, preferred_element_type=F32)
    total = cnt[:, M - 1:M]
    empty = jnp.where(total > 0.0, 0.0, 1.0)
    e0row = jnp.where(
        jax.lax.broadcasted_iota(jnp.int32, (1, M), 1) == 0, 1.0, 0.0)
    maskf = maskf + empty * e0row
    cnt = cnt + empty
    total = total + empty

    mci = (maskf * cnt).astype(jnp.int32)
    s_io1 = jax.lax.broadcasted_iota(jnp.int32, (1, S, 1), 1) + 1
    ohf = jnp.where(mci[:, None, :] == s_io1, 1.0, 0.0).astype(bf)
    G = ohf.reshape(P * S, M)

    cb = cxc * wx[0:1, :] + cyc * wx[1:2, :] + czc * wx[2:3, :]
    g = jnp.dot(G, A.astype(bf), preferred_element_type=F32).reshape(P, S, -1)
    g = g + (b1 - cb)[:, None, :]
    h = _relu(g).reshape(P * S, -1)
    h = _relu(jnp.dot(h.astype(bf), w2, preferred_element_type=F32) + b2)
    h = _relu(jnp.dot(h.astype(bf), w3, preferred_element_type=F32) + b3)
    C3 = h.shape[-1]
    pen = jnp.where(s_io1 <= total[:, :, None].astype(jnp.int32), 0.0, -1e30)
    return jnp.max(h.reshape(P, S, C3) + pen, axis=1)


def _k2_body(x_ref, y_ref, z_ref, xyz3_ref, feats_ref,
             c1m_ref, c1x_ref, c1y_ref, c1z_ref, c2m_ref,
             w1x, w1f, b11, w12, b12, w13, b13,
             w2x, w2f, b21, w22, b22, w23, b23,
             out_ref):
    feats = feats_ref[0]
    xyz3 = xyz3_ref[0]
    c1m = c1m_ref[0]
    c2m = c2m_ref[0]
    w2xv = w2x[...]

    A1 = (jnp.dot(xyz3, w1x[...], preferred_element_type=F32)
          + jnp.dot(feats, w1f[...], preferred_element_type=F32))
    f1 = _group_stage(x_ref[0], y_ref[0], z_ref[0], A1, c1m, 0.2 * 0.2, 64,
                      w1x[...], b11[...], w12[...], b12[...], w13[...], b13[...])

    A2 = (jnp.dot(c1m, w2xv, preferred_element_type=F32)
          + jnp.dot(f1, w2f[...], preferred_element_type=F32))
    f2 = _group_stage(c1x_ref[0], c1y_ref[0], c1z_ref[0], A2, c2m,
                      0.4 * 0.4, 64,
                      w2xv, b21[...], w22[...], b22[...], w23[...], b23[...])
    out_ref[...] = f2[None, :, :]


def _k3_body(xp_ref, fp_ref,
             w3x, w3f, b31, w32, b32, w33, b33,
             wc1, bc1, wc2, bc2, wc3, bc3,
             wr1, br1, wr2, br2, wr3, br3,
             out_ref, *, rois_per_block, npoint):
    w3xv = w3x[...]
    xp = xp_ref[...]
    h = _relu(xp[:, 0:1] * w3xv[0:1, :] + xp[:, 1:2] * w3xv[1:2, :]
              + xp[:, 2:3] * w3xv[2:3, :]
              + jnp.dot(fp_ref[...], w3f[...], preferred_element_type=F32)
              + b31[...])
    h = _relu(jnp.dot(h, w32[...], preferred_element_type=F32) + b32[...])
    h = _relu(jnp.dot(h, w33[...], preferred_element_type=F32) + b33[...])
    f3 = jnp.max(h.reshape(rois_per_block, npoint, h.shape[-1]), axis=1)

    hc = _relu(jnp.dot(f3, wc1[...], preferred_element_type=F32) + bc1[...])
    hc = _relu(jnp.dot(hc, wc2[...], preferred_element_type=F32) + bc2[...])
    c = jnp.dot(hc, wc3[...], preferred_element_type=F32) + bc3[...]

    hr = _relu(jnp.dot(f3, wr1[...], preferred_element_type=F32) + br1[...])
    hr = _relu(jnp.dot(hr, wr2[...], preferred_element_type=F32) + br2[...])
    r = jnp.dot(hr, wr3[...], preferred_element_type=F32) + br3[...]

    out_ref[:, 0:1] = c
    out_ref[:, 1:] = r


def _full_spec(shape):
    n = len(shape)
    return pl.BlockSpec(shape, lambda i, _n=n: (0,) * _n)


def _wt(wb):
    W, b = wb
    return W.T.astype(F32), b.reshape(1, -1).astype(F32)


def kernel(pts_input, roi_boxes3d, params):
    B, N, C = pts_input.shape

    bf = jnp.bfloat16
    w0, b0 = _wt(params['xyz_up'][0])
    w1, b1 = _wt(params['xyz_up'][1])
    Wm, bm_ = params['merge']
    wmh = Wm[:, :128].T
    wmr = Wm[:, 128:].T
    bm = bm_.reshape(1, -1)
    w0p = jnp.zeros((C, 128), F32).at[:5].set(w0).astype(bf)
    wmrp = jnp.zeros((C, 128), F32).at[5:].set(wmr).astype(bf)

    s1 = params['sa1']
    w1x = s1[0][0][:, :3].T
    w1f = s1[0][0][:, 3:].T
    b11 = s1[0][1].reshape(1, -1)
    w12, b12 = _wt(s1[1])
    w13, b13 = _wt(s1[2])

    s2 = params['sa2']
    w2x = s2[0][0][:, :3].T
    w2f = s2[0][0][:, 3:].T
    b21 = s2[0][1].reshape(1, -1)
    w22, b22 = _wt(s2[1])
    w23, b23 = _wt(s2[2])

    s3 = params['sa3']
    w3x = s3[0][0][:, :3].T
    w3f = s3[0][0][:, 3:].T
    b31 = s3[0][1].reshape(1, -1)
    w32, b32 = _wt(s3[1])
    w33, b33 = _wt(s3[2])

    wc1, bc1 = _wt(params['cls'][0])
    wc2, bc2 = _wt(params['cls'][1])
    wc3, bc3 = _wt(params['cls'][2])
    wr1, br1 = _wt(params['reg'][0])
    wr2, br2 = _wt(params['reg'][1])
    wr3, br3 = _wt(params['reg'][2])

    rows = B * N
    rpb0 = 4 if B % 4 == 0 else 1
    blk0 = rpb0 * N
    g0 = B // rpb0
    w1b, wmhb = w1.astype(bf), wmh.astype(bf)
    feats = pl.pallas_call(
        functools.partial(_k0_body, rows=blk0),
        grid=(g0,),
        in_specs=[
            pl.BlockSpec((rpb0, N, C), lambda i: (i, 0, 0)),
            _full_spec(w0p.shape), _full_spec(b0.shape),
            _full_spec(w1b.shape), _full_spec(b1.shape),
            _full_spec(wmhb.shape), _full_spec(wmrp.shape),
            _full_spec(bm.shape),
        ],
        out_specs=pl.BlockSpec((rpb0, N, 128), lambda i: (i, 0, 0)),
        out_shape=jax.ShapeDtypeStruct((B, N, 128), jnp.bfloat16),
    )(pts_input, w0p, b0, w1b, b1, wmhb, wmrp, bm)

    x = pts_input[:, :, 0]
    y = pts_input[:, :, 1]
    z = pts_input[:, :, 2]
    xyz3 = pts_input[:, :, 0:3]
    R = B
    g1 = B // R
    fps_out = pl.pallas_call(
        _k1_body,
        grid=(g1,),
        in_specs=[pl.BlockSpec((R, N), lambda i: (i, 0))] * 3,
        out_specs=[pl.BlockSpec((R, 128), lambda i: (i, 0))] * 3
        + [pl.BlockSpec((R, 32), lambda i: (i, 0))] * 3,
        out_shape=[jax.ShapeDtypeStruct((B, 128), F32)] * 3
        + [jax.ShapeDtypeStruct((B, 32), F32)] * 3,
    )(x, y, z)
    c1x, c1y, c1z, c2x, c2y, c2z = fps_out
    c1mat = jnp.stack([c1x, c1y, c1z], axis=-1)
    c2mat = jnp.stack([c2x, c2y, c2z], axis=-1)

    x3 = x.reshape(B, 1, N)
    y3 = y.reshape(B, 1, N)
    z3 = z.reshape(B, 1, N)
    c1x3 = c1x.reshape(B, 1, 128)
    c1y3 = c1y.reshape(B, 1, 128)
    c1z3 = c1z.reshape(B, 1, 128)

    w_in = [w1x, w1f.astype(bf), b11, w12.astype(bf), b12, w13.astype(bf), b13,
            w2x, w2f, b21, w22.astype(bf), b22, w23.astype(bf), b23]
    f2 = pl.pallas_call(
        _k2_body,
        grid=(B,),
        in_specs=[
            pl.BlockSpec((1, 1, N), lambda i: (i, 0, 0)),
            pl.BlockSpec((1, 1, N), lambda i: (i, 0, 0)),
            pl.BlockSpec((1, 1, N), lambda i: (i, 0, 0)),
            pl.BlockSpec((1, N, 3), lambda i: (i, 0, 0)),
            pl.BlockSpec((1, N, 128), lambda i: (i, 0, 0)),
            pl.BlockSpec((1, 128, 3), lambda i: (i, 0, 0)),
            pl.BlockSpec((1, 1, 128), lambda i: (i, 0, 0)),
            pl.BlockSpec((1, 1, 128), lambda i: (i, 0, 0)),
            pl.BlockSpec((1, 1, 128), lambda i: (i, 0, 0)),
            pl.BlockSpec((1, 32, 3), lambda i: (i, 0, 0)),
        ] + [_full_spec(w.shape) for w in w_in],
        out_specs=pl.BlockSpec((1, 32, 256), lambda i: (i, 0, 0)),
        out_shape=jax.ShapeDtypeStruct((B, 32, 256), F32),
    )(x3, y3, z3, xyz3, feats, c1mat, c1x3, c1y3, c1z3, c2mat, *w_in)

    xp = c2mat.reshape(B * 32, 3)
    fp = f2.reshape(B * 32, 256)
    rpb = 64 if B % 64 == 0 else B
    g3 = B // rpb
    w3_in = [w3x, w3f, b31, w32, b32, w33, b33,
             wc1, bc1, wc2, bc2, wc3, bc3,
             wr1, br1, wr2, br2, wr3, br3]
    out = pl.pallas_call(
        functools.partial(_k3_body, rois_per_block=rpb, npoint=32),
        grid=(g3,),
        in_specs=[
            pl.BlockSpec((rpb * 32, 3), lambda i: (i, 0)),
            pl.BlockSpec((rpb * 32, 256), lambda i: (i, 0)),
        ] + [_full_spec(w.shape) for w in w3_in],
        out_specs=pl.BlockSpec((rpb, 47), lambda i: (i, 0)),
        out_shape=jax.ShapeDtypeStruct((B, 47), F32),
    )(xp, fp, *w3_in)
    return out


# speedup vs baseline: 1.1182x; 1.0469x over previous
import functools

import jax
import jax.numpy as jnp
from jax.experimental import pallas as pl

F32 = jnp.float32


def _relu(x):
    return jnp.maximum(x, 0.0)


def _k0_body(ptsT_ref, w0pT, b0c, w1T, b1c, wmhT, wmrpT, bmc,
             out_ref, *, rois):
    bf = jnp.bfloat16
    for r in range(rois):
        pT = ptsT_ref[:, r, :]
        pT_bf = pT.astype(bf)
        hT = _relu(jnp.dot(w0pT[...], pT_bf, preferred_element_type=F32)
                   + b0c[...])
        hT = _relu(jnp.dot(w1T[...], hT.astype(bf),
                           preferred_element_type=F32) + b1c[...])
        fT = _relu(
            jnp.dot(wmhT[...], hT.astype(bf), preferred_element_type=F32)
            + jnp.dot(wmrpT[...], pT_bf, preferred_element_type=F32)
            + bmc[...]
        )
        out_ref[r] = jnp.transpose(fT.astype(bf), (1, 0))


def _fps_block(xa, ya, za, npoint):
    R, M = xa.shape
    lane = jax.lax.broadcasted_iota(jnp.int32, (R, M), 1)
    col = jax.lax.broadcasted_iota(jnp.int32, (R, npoint), 1)

    px = xa[:, 0:1]
    py = ya[:, 0:1]
    pz = za[:, 0:1]
    cx = jnp.where(col == 0, px, 0.0)
    cy = jnp.where(col == 0, py, 0.0)
    cz = jnp.where(col == 0, pz, 0.0)
    dists = jnp.full((R, M), 1e10, dtype=F32)

    def body(i, st):
        dists, px, py, pz, cx, cy, cz = st
        dx = xa - px
        dy = ya - py
        dz = za - pz
        d = (dx * dx + dy * dy) + dz * dz
        dists = jnp.minimum(dists, d)
        m = jnp.max(dists, axis=1, keepdims=True)
        idx = jnp.min(jnp.where(dists == m, lane, M), axis=1, keepdims=True)
        oh = lane == idx
        nx = jnp.sum(jnp.where(oh, xa, 0.0), axis=1, keepdims=True)
        ny = jnp.sum(jnp.where(oh, ya, 0.0), axis=1, keepdims=True)
        nz = jnp.sum(jnp.where(oh, za, 0.0), axis=1, keepdims=True)
        sel = col == i
        cx = jnp.where(sel, nx, cx)
        cy = jnp.where(sel, ny, cy)
        cz = jnp.where(sel, nz, cz)
        return (dists, nx, ny, nz, cx, cy, cz)

    st = jax.lax.fori_loop(1, npoint, body, (dists, px, py, pz, cx, cy, cz))
    return st[4], st[5], st[6]


def _k1_body(x_ref, y_ref, z_ref,
             c1x_ref, c1y_ref, c1z_ref, c2x_ref, c2y_ref, c2z_ref):
    c1x, c1y, c1z = _fps_block(x_ref[...], y_ref[...], z_ref[...], 128)
    c1x_ref[...] = c1x
    c1y_ref[...] = c1y
    c1z_ref[...] = c1z
    c2x, c2y, c2z = _fps_block(c1x, c1y, c1z, 32)
    c2x_ref[...] = c2x
    c2y_ref[...] = c2y
    c2z_ref[...] = c2z


def _group_stage(cxr, cyr, czr, A, cmat, r2, S,
                 wx, b1, w2, b2, w3, b3):
    P = cmat.shape[0]
    M = cxr.shape[1]
    cxc = cmat[:, 0:1]
    cyc = cmat[:, 1:2]
    czc = cmat[:, 2:3]
    dx = cxc - cxr
    dy = cyc - cyr
    dz = czc - czr
    d2 = (dx * dx + dy * dy) + dz * dz
    maskf = jnp.where(d2 < r2, 1.0, 0.0)

    io_r = jax.lax.broadcasted_iota(jnp.int32, (M, M), 0)
    io_c = jax.lax.broadcasted_iota(jnp.int32, (M, M), 1)
    bf = jnp.bfloat16
    tri = jnp.where(io_r <= io_c, 1.0, 0.0).astype(bf)
    cnt = jnp.dot(maskf.astype(bf), tri, preferred_element_type=F32)
    total = cnt[:, M - 1:M]
    empty = jnp.where(total > 0.0, 0.0, 1.0)
    e0row = jnp.where(
        jax.lax.broadcasted_iota(jnp.int32, (1, M), 1) == 0, 1.0, 0.0)
    maskf = maskf + empty * e0row
    cnt = cnt + empty
    total = total + empty

    mci = (maskf * cnt).astype(jnp.int32)
    s_io1 = jax.lax.broadcasted_iota(jnp.int32, (1, S, 1), 1) + 1
    ohf = jnp.where(mci[:, None, :] == s_io1, 1.0, 0.0).astype(bf)
    G = ohf.reshape(P * S, M)

    cb = cxc * wx[0:1, :] + cyc * wx[1:2, :] + czc * wx[2:3, :]
    g = jnp.dot(G, A.astype(bf), preferred_element_type=F32).reshape(P, S, -1)
    g = g + (b1 - cb)[:, None, :]
    h = _relu(g).reshape(P * S, -1)
    h = _relu(jnp.dot(h.astype(bf), w2, preferred_element_type=F32) + b2)
    h = _relu(jnp.dot(h.astype(bf), w3, preferred_element_type=F32) + b3)
    C3 = h.shape[-1]
    pen = jnp.where(s_io1 <= total[:, :, None].astype(jnp.int32), 0.0, -1e30)
    return jnp.max(h.reshape(P, S, C3) + pen, axis=1)


def _k2_body(x_ref, y_ref, z_ref, xyz3_ref, feats_ref,
             c1m_ref, c1x_ref, c1y_ref, c1z_ref, c2m_ref,
             w1x, w1f, b11, w12, b12, w13, b13,
             w2x, w2f, b21, w22, b22, w23, b23,
             out_ref):
    feats = feats_ref[0]
    xyz3 = xyz3_ref[0]
    c1m = c1m_ref[0]
    c2m = c2m_ref[0]
    w2xv = w2x[...]

    A1 = (jnp.dot(xyz3, w1x[...], preferred_element_type=F32)
          + jnp.dot(feats, w1f[...], preferred_element_type=F32))
    f1 = _group_stage(x_ref[0], y_ref[0], z_ref[0], A1, c1m, 0.2 * 0.2, 64,
                      w1x[...], b11[...], w12[...], b12[...], w13[...], b13[...])

    A2 = (jnp.dot(c1m, w2xv, preferred_element_type=F32)
          + jnp.dot(f1, w2f[...], preferred_element_type=F32))
    f2 = _group_stage(c1x_ref[0], c1y_ref[0], c1z_ref[0], A2, c2m,
                      0.4 * 0.4, 64,
                      w2xv, b21[...], w22[...], b22[...], w23[...], b23[...])
    out_ref[...] = f2[None, :, :]


def _k3_body(xp_ref, fp_ref,
             w3x, w3f, b31, w32, b32, w33, b33,
             wc1, bc1, wc2, bc2, wc3, bc3,
             wr1, br1, wr2, br2, wr3, br3,
             out_ref, *, rois_per_block, npoint):
    w3xv = w3x[...]
    xp = xp_ref[...]
    h = _relu(xp[:, 0:1] * w3xv[0:1, :] + xp[:, 1:2] * w3xv[1:2, :]
              + xp[:, 2:3] * w3xv[2:3, :]
              + jnp.dot(fp_ref[...], w3f[...], preferred_element_type=F32)
              + b31[...])
    h = _relu(jnp.dot(h, w32[...], preferred_element_type=F32) + b32[...])
    h = _relu(jnp.dot(h, w33[...], preferred_element_type=F32) + b33[...])
    f3 = jnp.max(h.reshape(rois_per_block, npoint, h.shape[-1]), axis=1)

    hc = _relu(jnp.dot(f3, wc1[...], preferred_element_type=F32) + bc1[...])
    hc = _relu(jnp.dot(hc, wc2[...], preferred_element_type=F32) + bc2[...])
    c = jnp.dot(hc, wc3[...], preferred_element_type=F32) + bc3[...]

    hr = _relu(jnp.dot(f3, wr1[...], preferred_element_type=F32) + br1[...])
    hr = _relu(jnp.dot(hr, wr2[...], preferred_element_type=F32) + br2[...])
    r = jnp.dot(hr, wr3[...], preferred_element_type=F32) + br3[...]

    out_ref[:, 0:1] = c
    out_ref[:, 1:] = r


def _full_spec(shape):
    n = len(shape)
    return pl.BlockSpec(shape, lambda i, _n=n: (0,) * _n)


def _wt(wb):
    W, b = wb
    return W.T.astype(F32), b.reshape(1, -1).astype(F32)


def kernel(pts_input, roi_boxes3d, params):
    B, N, C = pts_input.shape

    bf = jnp.bfloat16
    w0, b0 = _wt(params['xyz_up'][0])
    w1, b1 = _wt(params['xyz_up'][1])
    Wm, bm_ = params['merge']
    wmh = Wm[:, :128].T
    wmr = Wm[:, 128:].T
    bm = bm_.reshape(1, -1)
    w0p = jnp.zeros((C, 128), F32).at[:5].set(w0).astype(bf)
    wmrp = jnp.zeros((C, 128), F32).at[5:].set(wmr).astype(bf)

    s1 = params['sa1']
    w1x = s1[0][0][:, :3].T
    w1f = s1[0][0][:, 3:].T
    b11 = s1[0][1].reshape(1, -1)
    w12, b12 = _wt(s1[1])
    w13, b13 = _wt(s1[2])

    s2 = params['sa2']
    w2x = s2[0][0][:, :3].T
    w2f = s2[0][0][:, 3:].T
    b21 = s2[0][1].reshape(1, -1)
    w22, b22 = _wt(s2[1])
    w23, b23 = _wt(s2[2])

    s3 = params['sa3']
    w3x = s3[0][0][:, :3].T
    w3f = s3[0][0][:, 3:].T
    b31 = s3[0][1].reshape(1, -1)
    w32, b32 = _wt(s3[1])
    w33, b33 = _wt(s3[2])

    wc1, bc1 = _wt(params['cls'][0])
    wc2, bc2 = _wt(params['cls'][1])
    wc3, bc3 = _wt(params['cls'][2])
    wr1, br1 = _wt(params['reg'][0])
    wr2, br2 = _wt(params['reg'][1])
    wr3, br3 = _wt(params['reg'][2])

    rpb0 = 8 if B % 8 == 0 else B
    g0 = B // rpb0
    ptsT = jnp.transpose(pts_input, (2, 0, 1))
    w0pT = w0p.T
    w1T = w1.T.astype(bf)
    wmhT = wmh.T.astype(bf)
    wmrpT = wmrp.T
    b0c = b0.reshape(-1, 1)
    b1c = b1.reshape(-1, 1)
    bmc = bm.reshape(-1, 1)
    feats = pl.pallas_call(
        functools.partial(_k0_body, rois=rpb0),
        grid=(g0,),
        in_specs=[
            pl.BlockSpec((C, rpb0, N), lambda i: (0, i, 0)),
            _full_spec(w0pT.shape), _full_spec(b0c.shape),
            _full_spec(w1T.shape), _full_spec(b1c.shape),
            _full_spec(wmhT.shape), _full_spec(wmrpT.shape),
            _full_spec(bmc.shape),
        ],
        out_specs=pl.BlockSpec((rpb0, N, 128), lambda i: (i, 0, 0)),
        out_shape=jax.ShapeDtypeStruct((B, N, 128), jnp.bfloat16),
    )(ptsT, w0pT, b0c, w1T, b1c, wmhT, wmrpT, bmc)

    x = pts_input[:, :, 0]
    y = pts_input[:, :, 1]
    z = pts_input[:, :, 2]
    xyz3 = pts_input[:, :, 0:3]
    R = B
    g1 = B // R
    fps_out = pl.pallas_call(
        _k1_body,
        grid=(g1,),
        in_specs=[pl.BlockSpec((R, N), lambda i: (i, 0))] * 3,
        out_specs=[pl.BlockSpec((R, 128), lambda i: (i, 0))] * 3
        + [pl.BlockSpec((R, 32), lambda i: (i, 0))] * 3,
        out_shape=[jax.ShapeDtypeStruct((B, 128), F32)] * 3
        + [jax.ShapeDtypeStruct((B, 32), F32)] * 3,
    )(x, y, z)
    c1x, c1y, c1z, c2x, c2y, c2z = fps_out
    c1mat = jnp.stack([c1x, c1y, c1z], axis=-1)
    c2mat = jnp.stack([c2x, c2y, c2z], axis=-1)

    x3 = x.reshape(B, 1, N)
    y3 = y.reshape(B, 1, N)
    z3 = z.reshape(B, 1, N)
    c1x3 = c1x.reshape(B, 1, 128)
    c1y3 = c1y.reshape(B, 1, 128)
    c1z3 = c1z.reshape(B, 1, 128)

    w_in = [w1x, w1f.astype(bf), b11, w12.astype(bf), b12, w13.astype(bf), b13,
            w2x, w2f, b21, w22.astype(bf), b22, w23.astype(bf), b23]
    f2 = pl.pallas_call(
        _k2_body,
        grid=(B,),
        in_specs=[
            pl.BlockSpec((1, 1, N), lambda i: (i, 0, 0)),
            pl.BlockSpec((1, 1, N), lambda i: (i, 0, 0)),
            pl.BlockSpec((1, 1, N), lambda i: (i, 0, 0)),
            pl.BlockSpec((1, N, 3), lambda i: (i, 0, 0)),
            pl.BlockSpec((1, N, 128), lambda i: (i, 0, 0)),
            pl.BlockSpec((1, 128, 3), lambda i: (i, 0, 0)),
            pl.BlockSpec((1, 1, 128), lambda i: (i, 0, 0)),
            pl.BlockSpec((1, 1, 128), lambda i: (i, 0, 0)),
            pl.BlockSpec((1, 1, 128), lambda i: (i, 0, 0)),
            pl.BlockSpec((1, 32, 3), lambda i: (i, 0, 0)),
        ] + [_full_spec(w.shape) for w in w_in],
        out_specs=pl.BlockSpec((1, 32, 256), lambda i: (i, 0, 0)),
        out_shape=jax.ShapeDtypeStruct((B, 32, 256), F32),
    )(x3, y3, z3, xyz3, feats, c1mat, c1x3, c1y3, c1z3, c2mat, *w_in)

    xp = c2mat.reshape(B * 32, 3)
    fp = f2.reshape(B * 32, 256)
    rpb = 64 if B % 64 == 0 else B
    g3 = B // rpb
    w3_in = [w3x, w3f, b31, w32, b32, w33, b33,
             wc1, bc1, wc2, bc2, wc3, bc3,
             wr1, br1, wr2, br2, wr3, br3]
    out = pl.pallas_call(
        functools.partial(_k3_body, rois_per_block=rpb, npoint=32),
        grid=(g3,),
        in_specs=[
            pl.BlockSpec((rpb * 32, 3), lambda i: (i, 0)),
            pl.BlockSpec((rpb * 32, 256), lambda i: (i, 0)),
        ] + [_full_spec(w.shape) for w in w3_in],
        out_specs=pl.BlockSpec((rpb, 47), lambda i: (i, 0)),
        out_shape=jax.ShapeDtypeStruct((B, 47), F32),
    )(xp, fp, *w3_in)
    return out
